# Initial kernel scaffold; baseline (speedup 1.0000x reference)
import jax
import jax.numpy as jnp
from jax.experimental import pallas as pl


def kernel(x, eps, embeddings, W_enc0, b_enc0, W_enc1, b_enc1, W_mu, b_mu, W_lv, b_lv, W_dec, b_dec, W_dec0, b_dec0, W_dec1, b_dec1, W_dec2, b_dec2):
    raise NotImplementedError("write your pallas kernel here")


# speedup vs baseline: 1.6370x; 1.6370x over previous
import jax
import jax.numpy as jnp
from jax import lax
from jax.experimental import pallas as pl

SOM_X, SOM_Y = 16, 16
N_CODES = SOM_X * SOM_Y
LATENT = 64
BATCH = 1024


def _lrelu(x):
    return jnp.where(x > 0, x, 0.01 * x)


def _vae_body(x_ref, eps_ref, e_ref, et_ref,
              w0_ref, b0_ref, w1_ref, b1_ref, wm_ref, bm_ref, wl_ref, bl_ref,
              wd_ref, bd_ref, wd0_ref, bd0_ref, wd1_ref, bd1_ref,
              wd2_ref, bd2_ref,
              ze_o, zq_o, up_o, dn_o, lf_o, de_o, dq_o):
    x = x_ref[...]
    eps = eps_ref[...]

    h = _lrelu(x * w0_ref[...] + b0_ref[...])
    h = _lrelu(jnp.dot(h, w1_ref[...]) + b1_ref[...])
    mu = jnp.dot(h, wm_ref[...]) + bm_ref[...]
    logvar = jnp.dot(h, wl_ref[...]) + bl_ref[...]
    z = mu + eps * jnp.exp(0.5 * logvar)
    ze_o[...] = z

    d_acc = jnp.zeros((BATCH, N_CODES), jnp.float32)
    et = et_ref[...]
    for d in range(LATENT):
        diff = et[d:d + 1, :] - z[:, d:d + 1]
        d_acc = d_acc + diff * diff

    m = jnp.min(d_acc, axis=1, keepdims=True)
    iota = lax.broadcasted_iota(jnp.int32, (BATCH, N_CODES), 1)
    n = jnp.min(jnp.where(d_acc == m, iota, N_CODES * 2), axis=1)

    idx_up = n + SOM_Y
    idx_dn = n - SOM_Y
    idx_lf = jnp.where((n & (SOM_Y - 1)) > 0, n - 1, -1)

    e = e_ref[...]

    def gather(idx):
        oh = (iota == idx[:, None]).astype(jnp.float32)
        return jnp.dot(oh, e, precision=lax.Precision.HIGHEST)

    zq = gather(n)
    zq_o[...] = zq
    up_o[...] = gather(idx_up)
    dn_o[...] = gather(idx_dn)
    lf_o[...] = gather(idx_lf)

    def decode(zz):
        t = _lrelu(jnp.dot(zz, wd_ref[...]) + bd_ref[...])
        t = _lrelu(jnp.dot(t, wd0_ref[...]) + bd0_ref[...])
        t = _lrelu(jnp.dot(t, wd1_ref[...]) + bd1_ref[...])
        t = _lrelu(jnp.dot(t, wd2_ref[...]) + bd2_ref[...])
        return t

    de_o[...] = decode(z)
    dq_o[...] = decode(zq)


def kernel(x, eps, embeddings, W_enc0, b_enc0, W_enc1, b_enc1, W_mu, b_mu,
           W_lv, b_lv, W_dec, b_dec, W_dec0, b_dec0, W_dec1, b_dec1,
           W_dec2, b_dec2):
    e_flat = embeddings.reshape(N_CODES, LATENT)
    e_t = e_flat.T

    def row(b):
        return b.reshape(1, -1)

    f32 = jnp.float32
    outs = pl.pallas_call(
        _vae_body,
        out_shape=[
            jax.ShapeDtypeStruct((BATCH, LATENT), f32),
            jax.ShapeDtypeStruct((BATCH, LATENT), f32),
            jax.ShapeDtypeStruct((BATCH, LATENT), f32),
            jax.ShapeDtypeStruct((BATCH, LATENT), f32),
            jax.ShapeDtypeStruct((BATCH, LATENT), f32),
            jax.ShapeDtypeStruct((BATCH, 1), f32),
            jax.ShapeDtypeStruct((BATCH, 1), f32),
        ],
    )(x, eps, e_flat, e_t,
      row(W_enc0.T.reshape(-1)), row(b_enc0), W_enc1.T, row(b_enc1),
      W_mu.T, row(b_mu), W_lv.T, row(b_lv),
      W_dec.T, row(b_dec), W_dec0.T, row(b_dec0), W_dec1.T, row(b_dec1),
      W_dec2.T, row(b_dec2))

    z_e, z_q, up, dn, lf, de, dq = outs
    z_q_neighbors = jnp.stack([z_q, up, dn, jnp.zeros_like(z_q), lf], axis=1)
    return (z_e, z_q, z_q_neighbors, de, dq)


# speedup vs baseline: 2.0090x; 1.2273x over previous
import jax
import jax.numpy as jnp
from jax import lax
from jax.experimental import pallas as pl

SOM_X, SOM_Y = 16, 16
N_CODES = SOM_X * SOM_Y
LATENT = 64
BATCH = 1024
_HI = lax.Precision.HIGHEST


def _lrelu(x):
    return jnp.where(x > 0, x, 0.01 * x)


def _vae_body(x_ref, eps_ref, ecat_ref, et_ref,
              w0_ref, b0_ref, w1_ref, b1_ref, wm_ref, bm_ref, wl_ref, bl_ref,
              wd_ref, bd_ref, wd0_ref, bd0_ref, wd1_ref, bd1_ref,
              wd2_ref, bd2_ref,
              ze_o, zq_o, up_o, dn_o, lf_o, de_o, dq_o):
    x = x_ref[...]
    eps = eps_ref[...]

    h = _lrelu(x * w0_ref[...] + b0_ref[...])
    h = _lrelu(jnp.dot(h, w1_ref[...]) + b1_ref[...])
    mu = jnp.dot(h, wm_ref[...]) + bm_ref[...]
    logvar = jnp.dot(h, wl_ref[...]) + bl_ref[...]
    z = mu + eps * jnp.exp(0.5 * logvar)
    ze_o[...] = z

    et = et_ref[...]
    eb2 = jnp.sum(et * et, axis=0, keepdims=True)
    s = eb2 - 2.0 * jnp.dot(z, et, precision=_HI)

    iota = lax.broadcasted_iota(jnp.int32, (BATCH, N_CODES), 1)
    big = jnp.float32(3.4e38)

    m1 = jnp.min(s, axis=1, keepdims=True)
    n1 = jnp.min(jnp.where(s == m1, iota, N_CODES * 2), axis=1)
    s2 = jnp.where(iota == n1[:, None], big, s)
    m2 = jnp.min(s2, axis=1, keepdims=True)
    n2 = jnp.min(jnp.where(s2 == m2, iota, N_CODES * 2), axis=1)

    ecat = ecat_ref[...]
    e_base = ecat[:, :LATENT]

    def onehot(idx):
        return (iota == idx[:, None]).astype(jnp.float32)

    e1 = jnp.dot(onehot(n1), e_base, precision=_HI)
    e2 = jnp.dot(onehot(n2), e_base, precision=_HI)
    d1 = jnp.sum((e1 - z) * (e1 - z), axis=1)
    d2 = jnp.sum((e2 - z) * (e2 - z), axis=1)
    take2 = (d2 < d1) | ((d2 == d1) & (n2 < n1))
    n = jnp.where(take2, n2, n1)

    g = jnp.dot(onehot(n), ecat, precision=_HI)
    zq_o[...] = jnp.where(take2[:, None], e2, e1)
    up_o[...] = g[:, LATENT:2 * LATENT]
    dn_o[...] = g[:, 2 * LATENT:3 * LATENT]
    lf_o[...] = g[:, 3 * LATENT:]

    def decode(zz):
        t = _lrelu(jnp.dot(zz, wd_ref[...]) + bd_ref[...])
        t = _lrelu(jnp.dot(t, wd0_ref[...]) + bd0_ref[...])
        t = _lrelu(jnp.dot(t, wd1_ref[...]) + bd1_ref[...])
        t = _lrelu(jnp.dot(t, wd2_ref[...]) + bd2_ref[...])
        return t

    de_o[...] = decode(z)
    dq_o[...] = decode(jnp.where(take2[:, None], e2, e1))


def kernel(x, eps, embeddings, W_enc0, b_enc0, W_enc1, b_enc1, W_mu, b_mu,
           W_lv, b_lv, W_dec, b_dec, W_dec0, b_dec0, W_dec1, b_dec1,
           W_dec2, b_dec2):
    e_flat = embeddings.reshape(N_CODES, LATENT)
    e_t = e_flat.T
    zero16 = jnp.zeros((SOM_Y, LATENT), jnp.float32)
    e_up = jnp.concatenate([e_flat[SOM_Y:], zero16], axis=0)
    e_dn = jnp.concatenate([zero16, e_flat[:-SOM_Y]], axis=0)
    e_lf = jnp.concatenate([jnp.zeros((1, LATENT), jnp.float32),
                            e_flat[:-1]], axis=0)
    col_mask = (jnp.arange(N_CODES) % SOM_Y > 0).astype(jnp.float32)
    e_lf = e_lf * col_mask[:, None]
    e_cat = jnp.concatenate([e_flat, e_up, e_dn, e_lf], axis=1)

    def row(b):
        return b.reshape(1, -1)

    f32 = jnp.float32
    outs = pl.pallas_call(
        _vae_body,
        out_shape=[
            jax.ShapeDtypeStruct((BATCH, LATENT), f32),
            jax.ShapeDtypeStruct((BATCH, LATENT), f32),
            jax.ShapeDtypeStruct((BATCH, LATENT), f32),
            jax.ShapeDtypeStruct((BATCH, LATENT), f32),
            jax.ShapeDtypeStruct((BATCH, LATENT), f32),
            jax.ShapeDtypeStruct((BATCH, 1), f32),
            jax.ShapeDtypeStruct((BATCH, 1), f32),
        ],
    )(x, eps, e_cat, e_t,
      row(W_enc0.T.reshape(-1)), row(b_enc0), W_enc1.T, row(b_enc1),
      W_mu.T, row(b_mu), W_lv.T, row(b_lv),
      W_dec.T, row(b_dec), W_dec0.T, row(b_dec0), W_dec1.T, row(b_dec1),
      W_dec2.T, row(b_dec2))

    z_e, z_q, up, dn, lf, de, dq = outs
    z_q_neighbors = jnp.stack([z_q, up, dn, jnp.zeros_like(z_q), lf], axis=1)
    return (z_e, z_q, z_q_neighbors, de, dq)
